# Initial kernel scaffold; baseline (speedup 1.0000x reference)
import functools

import jax
import jax.numpy as jnp
from jax import lax
from jax.experimental import pallas as pl
from jax.experimental.pallas import tpu as pltpu
from jax.experimental.pallas import tpu_sc as plsc

N = 10000
E = 320000
D = 128

NPAD = 10240
NC, NS, L = 2, 16, 16
NW = NC * NS
KJ = 79
EPW = KJ * 128
EPAD = NW * EPW

ROW_BLK = 2048


def _stage_a_body(x_ref, w1t_ref, b1_ref, w2_ref, s_ref):
    h = jnp.dot(x_ref[...], w1t_ref[...], preferred_element_type=jnp.float32)
    h = jnp.sin(h + b1_ref[...])
    s = jnp.sum(h * w2_ref[...], axis=1)
    s_ref[...] = s.reshape(ROW_BLK // 128, 128)


def _stage_a(x_pad, w1t, b1, w2):
    grid = NPAD // ROW_BLK
    return pl.pallas_call(
        _stage_a_body,
        grid=(grid,),
        in_specs=[
            pl.BlockSpec((ROW_BLK, 2 * D), lambda i: (i, 0)),
            pl.BlockSpec((2 * D, D), lambda i: (0, 0)),
            pl.BlockSpec((1, D), lambda i: (0, 0)),
            pl.BlockSpec((1, D), lambda i: (0, 0)),
        ],
        out_specs=pl.BlockSpec((ROW_BLK // 128, 128), lambda i: (i, 0)),
        out_shape=jax.ShapeDtypeStruct((NPAD // 128, 128), jnp.float32),
    )(x_pad, w1t, b1, w2)


def _stage_b_body(s_hbm, col_hbm, row_hbm, w_hbm, out_hbm,
                  s_v, col_v, row2d, w_v, vals, zbuf, t_sh):
    cid = lax.axis_index("c")
    sid = lax.axis_index("s")
    wid = cid * NS + sid
    npt = NPAD // NS

    def zero_body(i, _):
        zbuf[pl.ds(i * L, L)] = jnp.zeros((L,), jnp.float32)
        return 0
    lax.fori_loop(0, npt // L, zero_body, 0)
    pltpu.sync_copy(zbuf, t_sh.at[pl.ds(sid * npt, npt)])

    pltpu.sync_copy(s_hbm, s_v)
    pltpu.sync_copy(col_hbm.at[pl.ds(wid * EPW, EPW)], col_v)
    pltpu.sync_copy(w_hbm.at[pl.ds(wid * EPW, EPW)], w_v)
    pltpu.sync_copy(row_hbm.at[wid], row2d)

    def gather_body(i, _):
        idx = col_v[pl.ds(i * L, L)]
        g = plsc.load_gather(s_v, [idx])
        vals[pl.ds(i * L, L)] = g * w_v[pl.ds(i * L, L)]
        return 0
    lax.fori_loop(0, EPW // L, gather_body, 0)

    plsc.subcore_barrier()

    def scat_body(j, _):
        pltpu.sync_copy(vals.at[pl.ds(j * 128, 128)],
                        t_sh.at[row2d.at[j]], add=True)
        return 0
    lax.fori_loop(0, KJ, scat_body, 0)

    plsc.subcore_barrier()

    pltpu.sync_copy(t_sh.at[pl.ds(sid * npt, npt)],
                    out_hbm.at[cid, pl.ds(sid * npt, npt)])


def _stage_b(s_flat, col_p, row3d, w_p):
    mesh = plsc.VectorSubcoreMesh(core_axis_name="c", subcore_axis_name="s")
    kern = pl.kernel(
        _stage_b_body,
        out_type=jax.ShapeDtypeStruct((NC, NPAD), jnp.float32),
        mesh=mesh,
        scratch_types=[
            pltpu.VMEM((NPAD,), jnp.float32),
            pltpu.VMEM((EPW,), jnp.int32),
            pltpu.VMEM((KJ, 128), jnp.int32),
            pltpu.VMEM((EPW,), jnp.float32),
            pltpu.VMEM((EPW,), jnp.float32),
            pltpu.VMEM((NPAD // NS,), jnp.float32),
            pltpu.VMEM_SHARED((NPAD,), jnp.float32),
        ],
    )
    return kern(s_flat, col_p, row3d, w_p)


def _stage_c_body(part_ref, b2_ref, o_ref):
    p = part_ref[...]
    t = p[0:1, :] + p[1:2, :]
    out2 = jnp.sin(t + b2_ref[0, 0])
    msk = lax.broadcasted_iota(jnp.int32, (1, NPAD), 1) < N
    sq = jnp.where(msk, out2 * out2, 0.0)
    o_ref[...] = jnp.sqrt(jnp.sum(sq)).reshape(1, 1)


def _stage_c(part, b2):
    return pl.pallas_call(
        _stage_c_body,
        out_shape=jax.ShapeDtypeStruct((1, 1), jnp.float32),
    )(part, b2.reshape(1, 1))


@jax.jit
def kernel(x, edge_index, edge_weight, W1, b1, W2, b2):
    x_pad = jnp.pad(x, ((0, NPAD - N), (0, 0)))
    s2d = _stage_a(x_pad, W1.T, b1.reshape(1, D), W2.reshape(1, D))
    s_flat = s2d.reshape(NPAD)

    row = edge_index[0].astype(jnp.int32)
    col = edge_index[1].astype(jnp.int32)
    pad = EPAD - E
    row3d = jnp.pad(row, (0, pad)).reshape(NW, KJ, 128)
    col_p = jnp.pad(col, (0, pad))
    w_p = jnp.pad(edge_weight, (0, pad))

    part = _stage_b(s_flat, col_p, row3d, w_p)
    out = _stage_c(part, b2)
    return out.reshape(1)


# speedup vs baseline: 23.9121x; 23.9121x over previous
import functools

import jax
import jax.numpy as jnp
from jax import lax
from jax.experimental import pallas as pl
from jax.experimental.pallas import tpu as pltpu
from jax.experimental.pallas import tpu_sc as plsc

N = 10000
E = 320000
D = 128

NPAD = 10240
NC, NS, L = 2, 16, 16
NW = NC * NS
KJ = 79
EPW = KJ * 128
EPAD = NW * EPW

ROW_BLK = 2048


def _stage_a_body(x_ref, w1t_ref, b1_ref, w2_ref, s_ref):
    h = jnp.dot(x_ref[...], w1t_ref[...], preferred_element_type=jnp.float32)
    h = jnp.sin(h + b1_ref[...])
    s = jnp.sum(h * w2_ref[...], axis=1)
    s_ref[...] = s.reshape(ROW_BLK // 128, 128)


def _stage_a(x_pad, w1t, b1, w2):
    grid = NPAD // ROW_BLK
    return pl.pallas_call(
        _stage_a_body,
        grid=(grid,),
        in_specs=[
            pl.BlockSpec((ROW_BLK, 2 * D), lambda i: (i, 0)),
            pl.BlockSpec((2 * D, D), lambda i: (0, 0)),
            pl.BlockSpec((1, D), lambda i: (0, 0)),
            pl.BlockSpec((1, D), lambda i: (0, 0)),
        ],
        out_specs=pl.BlockSpec((ROW_BLK // 128, 128), lambda i: (i, 0)),
        out_shape=jax.ShapeDtypeStruct((NPAD // 128, 128), jnp.float32),
    )(x_pad, w1t, b1, w2)


def _stage_b_body(s_hbm, col_hbm, row_hbm, w_hbm, out_hbm,
                  s_v, col_v, row2d, w_v, vals, zbuf, t_sh):
    cid = lax.axis_index("c")
    sid = lax.axis_index("s")
    wid = cid * NS + sid
    npt = NPAD // NS

    def zero_body(i, _):
        zbuf[pl.ds(i * L, L)] = jnp.zeros((L,), jnp.float32)
        return 0
    lax.fori_loop(0, npt // L, zero_body, 0)
    pltpu.sync_copy(zbuf, t_sh.at[pl.ds(sid * npt, npt)])

    pltpu.sync_copy(s_hbm, s_v)
    pltpu.sync_copy(col_hbm.at[pl.ds(wid * EPW, EPW)], col_v)
    pltpu.sync_copy(w_hbm.at[pl.ds(wid * EPW, EPW)], w_v)
    pltpu.sync_copy(row_hbm.at[wid], row2d)

    def gather_body(i, _):
        idx = col_v[pl.ds(i * L, L)]
        g = plsc.load_gather(s_v, [idx])
        vals[pl.ds(i * L, L)] = g * w_v[pl.ds(i * L, L)]
        return 0
    lax.fori_loop(0, EPW // L, gather_body, 0)

    plsc.subcore_barrier()

    def scat_body(j, _):
        pltpu.sync_copy(vals.at[pl.ds(j * 128, 128)],
                        t_sh.at[row2d.at[j]], add=True)
        return 0
    lax.fori_loop(0, KJ, scat_body, 0)

    plsc.subcore_barrier()

    pltpu.sync_copy(t_sh.at[pl.ds(sid * npt, npt)],
                    out_hbm.at[cid, pl.ds(sid * npt, npt)])


def _stage_b(s_flat, col_p, row3d, w_p):
    mesh = plsc.VectorSubcoreMesh(core_axis_name="c", subcore_axis_name="s")
    kern = pl.kernel(
        _stage_b_body,
        out_type=jax.ShapeDtypeStruct((NC, NPAD), jnp.float32),
        mesh=mesh,
        compiler_params=pltpu.CompilerParams(needs_layout_passes=False),
        scratch_types=[
            pltpu.VMEM((NPAD,), jnp.float32),
            pltpu.VMEM((EPW,), jnp.int32),
            pltpu.VMEM((KJ, 128), jnp.int32),
            pltpu.VMEM((EPW,), jnp.float32),
            pltpu.VMEM((EPW,), jnp.float32),
            pltpu.VMEM((NPAD // NS,), jnp.float32),
            pltpu.VMEM_SHARED((NPAD,), jnp.float32),
        ],
    )
    return kern(s_flat, col_p, row3d, w_p)


def _stage_c_body(part_ref, b2_ref, o_ref):
    p = part_ref[...]
    t = p[0:1, :] + p[1:2, :]
    out2 = jnp.sin(t + b2_ref[0, 0])
    msk = lax.broadcasted_iota(jnp.int32, (1, NPAD), 1) < N
    sq = jnp.where(msk, out2 * out2, 0.0)
    o_ref[...] = jnp.sqrt(jnp.sum(sq)).reshape(1, 1)


def _stage_c(part, b2):
    return pl.pallas_call(
        _stage_c_body,
        out_shape=jax.ShapeDtypeStruct((1, 1), jnp.float32),
    )(part, b2.reshape(1, 1))


@jax.jit
def kernel(x, edge_index, edge_weight, W1, b1, W2, b2):
    x_pad = jnp.pad(x, ((0, NPAD - N), (0, 0)))
    s2d = _stage_a(x_pad, W1.T, b1.reshape(1, D), W2.reshape(1, D))
    s_flat = s2d.reshape(NPAD)

    row = edge_index[0].astype(jnp.int32)
    col = edge_index[1].astype(jnp.int32)
    pad = EPAD - E
    row3d = jnp.pad(row, (0, pad)).reshape(NW, KJ, 128)
    col_p = jnp.pad(col, (0, pad))
    w_p = jnp.pad(edge_weight, (0, pad))

    part = _stage_b(s_flat, col_p, row3d, w_p)
    out = _stage_c(part, b2)
    return out.reshape(1)
